# Initial kernel scaffold; baseline (speedup 1.0000x reference)
import jax
import jax.numpy as jnp
from jax.experimental import pallas as pl


def kernel(input):
    raise NotImplementedError("write your pallas kernel here")


# speedup vs baseline: 6.8097x; 6.8097x over previous
import functools
import math

import numpy as np
import jax
import jax.numpy as jnp
from jax import lax
from jax.experimental import pallas as pl
from jax.experimental.pallas import tpu as pltpu
from jax.experimental.pallas import tpu_sc as plsc

_ROWS, _COLS = 64, 8192
_N = _ROWS * _COLS
_GOAL = math.floor(0.05 * _N)
_NT = 16
_CHUNK = _N // _NT
_NB = 128
_BLK = _N // _NB
_LPB = _BLK // 128


def _np_threefry2x32(k1, k2, x0, x1):
    x0 = x0.astype(np.uint32).copy()
    x1 = x1.astype(np.uint32).copy()
    ks = [np.uint32(k1), np.uint32(k2),
          np.uint32(k1) ^ np.uint32(k2) ^ np.uint32(0x1BD11BDA)]
    rot_a = (13, 15, 26, 6)
    rot_b = (17, 29, 16, 24)

    def rounds(x0, x1, rots):
        for r in rots:
            x0 = x0 + x1
            x1 = (x1 << np.uint32(r)) | (x1 >> np.uint32(32 - r))
            x1 = x0 ^ x1
        return x0, x1

    x0 += ks[0]
    x1 += ks[1]
    sched = ((rot_a, 1, 2), (rot_b, 2, 0), (rot_a, 0, 1),
             (rot_b, 1, 2), (rot_a, 2, 0))
    for i, (rots, a, b) in enumerate(sched):
        x0, x1 = rounds(x0, x1, rots)
        x0 += ks[a]
        x1 += ks[b] + np.uint32(i + 1)
    return x0, x1


def _np_uniform_key42(n):
    counts1 = np.zeros(n, np.uint32)
    counts2 = np.arange(n, dtype=np.uint32)
    bits1, bits2 = _np_threefry2x32(0, 42, counts1, counts2)
    bits = bits1 ^ bits2
    float_bits = (bits >> np.uint32(9)) | np.uint32(0x3F800000)
    floats = float_bits.view(np.float32) - np.float32(1.0)
    return np.maximum(np.float32(0.0), floats)


def _build_consts():
    u = _np_uniform_key42(_N)
    order = np.argsort(-u).astype(np.int32)
    su = u[order]
    rank = np.empty(_N, dtype=np.int64)
    rank[order] = np.arange(_N)
    bid = (rank // _BLK).astype(np.int32)
    return u, order, su, bid


_U_NP, _PERM_NP, _SU_NP, _BID_NP = _build_consts()
_U2_NP = np.ascontiguousarray(_U_NP.reshape(_ROWS, _COLS))
_PERM3_NP = np.ascontiguousarray(_PERM_NP.reshape(_NB, _LPB, 128))
_SU2_NP = np.ascontiguousarray(_SU_NP.reshape(_NB, _BLK))


def _sc_select(xf, bid, perm3, su2):
    mesh = plsc.VectorSubcoreMesh(core_axis_name="c", subcore_axis_name="s",
                                  num_cores=1)

    _HW = _NB * 16

    @functools.partial(
        pl.kernel,
        out_type=jax.ShapeDtypeStruct((16,), jnp.float32),
        mesh=mesh,
        compiler_params=pltpu.CompilerParams(needs_layout_passes=False),
        scratch_types=[
            pltpu.VMEM((_CHUNK,), jnp.float32),
            pltpu.VMEM((_CHUNK,), jnp.int32),
            pltpu.VMEM((_HW,), jnp.int32),
            pltpu.VMEM((_HW,), jnp.int32),
            pltpu.VMEM_SHARED((_NT, _HW), jnp.int32),
            pltpu.VMEM((_LPB, 128), jnp.int32),
            pltpu.VMEM((_BLK,), jnp.float32),
            pltpu.VMEM((_LPB, 128), jnp.float32),
            pltpu.VMEM((16,), jnp.float32),
            pltpu.SemaphoreType.DMA,
        ],
    )
    def k(x_hbm, bid_hbm, perm_hbm, su_hbm, out_hbm,
          xv, bv, histv, accv, shv, pv, suv, gv, kv, sem):
        wid = lax.axis_index("s")
        lanes = lax.iota(jnp.int32, 16)
        zeros16 = jnp.zeros((16,), jnp.int32)
        ones16 = jnp.ones((16,), jnp.int32)

        def zb(t, c):
            histv[pl.ds(t * 16, 16)] = zeros16
            return c
        lax.fori_loop(0, _HW // 16, zb, 0)

        base = wid * _CHUNK
        pltpu.sync_copy(x_hbm.at[pl.ds(base, _CHUNK)], xv)
        pltpu.sync_copy(bid_hbm.at[pl.ds(base, _CHUNK)], bv)

        def s1(i, c):
            xvec = xv[pl.ds(i * 16, 16)]
            bvec = bv[pl.ds(i * 16, 16)]
            m = xvec != 0.0
            plsc.addupdate_scatter(histv, [bvec * 16 + lanes], ones16, mask=m)
            return c
        lax.fori_loop(0, _CHUNK // 16, s1, 0)

        pltpu.sync_copy(histv, shv.at[wid])
        plsc.subcore_barrier()

        @pl.when(wid == 0)
        def _():
            for t in range(1, _NT):
                pltpu.sync_copy(shv.at[t], accv)

                def addrow(i, c):
                    histv[pl.ds(i * 16, 16)] = (histv[pl.ds(i * 16, 16)]
                                                + accv[pl.ds(i * 16, 16)])
                    return c
                lax.fori_loop(0, _HW // 16, addrow, 0)

            running = jnp.int32(0)
            found = jnp.int32(0)
            bstar = jnp.int32(0)
            cumbefore = jnp.int32(0)
            for c in range(_NB // 16):
                acc = zeros16
                for j in range(16):
                    acc = acc + plsc.load_gather(
                        histv, [c * 256 + lanes * 16 + j])
                s = plsc.cumsum(acc)
                cum = s + running
                mask = cum >= _GOAL
                hasn = jnp.max(plsc.all_reduce_population_count(mask))
                ffs = jnp.max(plsc.all_reduce_ffs(mask))
                hit = (found == 0) & (hasn > 0)
                prev = cum - acc
                prevsel = jnp.max(jnp.where(lanes == ffs, prev, jnp.int32(-1)))
                bstar = jnp.where(hit, c * 16 + ffs, bstar)
                cumbefore = jnp.where(hit, prevsel, cumbefore)
                found = jnp.where(hit, jnp.int32(1), found)
                running = jnp.max(cum)
            count_nz = running
            kprime = _GOAL - cumbefore
            prune = count_nz > _GOAL

            @pl.when(prune)
            def _():
                pltpu.sync_copy(perm_hbm.at[bstar], pv)
                pltpu.sync_copy(su_hbm.at[bstar], suv)
                for g in range(_LPB // 8):
                    hs = [pltpu.async_copy(x_hbm.at[pv.at[8 * g + j]],
                                           gv.at[8 * g + j], sem)
                          for j in range(8)]
                    for h in hs:
                        h.wait()

                def s3(i, carry):
                    cum, kth = carry
                    xg = gv[i // 8, pl.ds((i % 8) * 16, 16)]
                    sv = suv[pl.ds(i * 16, 16)]
                    m = xg != 0.0
                    s = plsc.cumsum(jnp.where(m, 1, 0).astype(jnp.int32))
                    sel = m & ((s + cum) == kprime)
                    cand = jnp.max(jnp.where(sel, sv, jnp.float32(-1.0)))
                    return (cum + jnp.max(s), jnp.maximum(kth, cand))
                _, kth = lax.fori_loop(0, _BLK // 16, s3,
                                       (jnp.int32(0), jnp.float32(-1.0)))
                kv[...] = jnp.zeros((16,), jnp.float32) + kth
                pltpu.sync_copy(kv, out_hbm)

            @pl.when(jnp.logical_not(prune))
            def _():
                kv[...] = jnp.full((16,), -1.0, jnp.float32)
                pltpu.sync_copy(kv, out_hbm)

    return k(xf, bid, perm3, su2)


def _tc_mask(kth11, x, u2):
    def body(kth_ref, x_ref, u_ref, o_ref):
        kth = kth_ref[0, 0]
        xb = x_ref[...]
        o_ref[...] = jnp.where((xb != 0.0) & (u_ref[...] >= kth), xb, 0.0)

    return pl.pallas_call(
        body,
        out_shape=jax.ShapeDtypeStruct((_ROWS, _COLS), jnp.float32),
        in_specs=[
            pl.BlockSpec(memory_space=pltpu.SMEM),
            pl.BlockSpec(memory_space=pltpu.VMEM),
            pl.BlockSpec(memory_space=pltpu.VMEM),
        ],
        out_specs=pl.BlockSpec(memory_space=pltpu.VMEM),
    )(kth11, x, u2)


def kernel(input):
    x = input
    xf = x.reshape(-1)
    kth16 = _sc_select(xf, jnp.asarray(_BID_NP), jnp.asarray(_PERM3_NP),
                       jnp.asarray(_SU2_NP))
    kth11 = kth16[:1].reshape(1, 1)
    return _tc_mask(kth11, x, jnp.asarray(_U2_NP))


# speedup vs baseline: 8.3412x; 1.2249x over previous
import functools
import math

import numpy as np
import jax
import jax.numpy as jnp
from jax import lax
from jax.experimental import pallas as pl
from jax.experimental.pallas import tpu as pltpu
from jax.experimental.pallas import tpu_sc as plsc

_ROWS, _COLS = 64, 8192
_N = _ROWS * _COLS
_GOAL = math.floor(0.05 * _N)
_NT = 16
_CHUNK = _N // _NT
_NB = 128
_BLK = _N // _NB
_LPB = _BLK // 128


def _np_threefry2x32(k1, k2, x0, x1):
    x0 = x0.astype(np.uint32).copy()
    x1 = x1.astype(np.uint32).copy()
    ks = [np.uint32(k1), np.uint32(k2),
          np.uint32(k1) ^ np.uint32(k2) ^ np.uint32(0x1BD11BDA)]
    rot_a = (13, 15, 26, 6)
    rot_b = (17, 29, 16, 24)

    def rounds(x0, x1, rots):
        for r in rots:
            x0 = x0 + x1
            x1 = (x1 << np.uint32(r)) | (x1 >> np.uint32(32 - r))
            x1 = x0 ^ x1
        return x0, x1

    x0 += ks[0]
    x1 += ks[1]
    sched = ((rot_a, 1, 2), (rot_b, 2, 0), (rot_a, 0, 1),
             (rot_b, 1, 2), (rot_a, 2, 0))
    for i, (rots, a, b) in enumerate(sched):
        x0, x1 = rounds(x0, x1, rots)
        x0 += ks[a]
        x1 += ks[b] + np.uint32(i + 1)
    return x0, x1


def _np_uniform_key42(n):
    counts1 = np.zeros(n, np.uint32)
    counts2 = np.arange(n, dtype=np.uint32)
    bits1, bits2 = _np_threefry2x32(0, 42, counts1, counts2)
    bits = bits1 ^ bits2
    float_bits = (bits >> np.uint32(9)) | np.uint32(0x3F800000)
    floats = float_bits.view(np.float32) - np.float32(1.0)
    return np.maximum(np.float32(0.0), floats)


def _build_consts():
    u = _np_uniform_key42(_N)
    order = np.argsort(-u).astype(np.int32)
    su = u[order]
    rank = np.empty(_N, dtype=np.int64)
    rank[order] = np.arange(_N)
    bid = (rank // _BLK).astype(np.int32)
    return u, order, su, bid


_U_NP, _PERM_NP, _SU_NP, _BID_NP = _build_consts()
_U2_NP = np.ascontiguousarray(_U_NP.reshape(_ROWS, _COLS))
_PERM3_NP = np.ascontiguousarray(_PERM_NP.reshape(_NB, _LPB, 128))
_SU2_NP = np.ascontiguousarray(_SU_NP.reshape(_NB, _BLK))


def _sc_select(xf, bid, perm3, su2):
    mesh = plsc.VectorSubcoreMesh(core_axis_name="c", subcore_axis_name="s",
                                  num_cores=1)

    _HW = _NB * 16
    _H = _CHUNK // 2

    @functools.partial(
        pl.kernel,
        out_type=jax.ShapeDtypeStruct((16,), jnp.float32),
        mesh=mesh,
        compiler_params=pltpu.CompilerParams(needs_layout_passes=False),
        scratch_types=[
            pltpu.VMEM((2, _H), jnp.float32),
            pltpu.VMEM((2, _H), jnp.int32),
            pltpu.VMEM((_HW,), jnp.int32),
            pltpu.VMEM((_NT, 128), jnp.int32),
            pltpu.VMEM((128,), jnp.int32),
            pltpu.VMEM((_HW,), jnp.int32),
            pltpu.VMEM_SHARED((_NT, _NT, 128), jnp.int32),
            pltpu.VMEM_SHARED((_HW,), jnp.int32),
            pltpu.VMEM((_LPB, 128), jnp.int32),
            pltpu.VMEM((_BLK,), jnp.float32),
            pltpu.VMEM((_LPB, 128), jnp.float32),
            pltpu.VMEM((16,), jnp.float32),
            pltpu.SemaphoreType.DMA,
            pltpu.SemaphoreType.DMA,
        ],
    )
    def k(x_hbm, bid_hbm, perm_hbm, su_hbm, out_hbm,
          xv, bv, histv, colv, sumv, hv2, shv3, shv2, pv, suv, gv, kv,
          sem0, sem1):
        wid = lax.axis_index("s")
        lanes = lax.iota(jnp.int32, 16)
        zeros16 = jnp.zeros((16,), jnp.int32)
        ones16 = jnp.ones((16,), jnp.int32)

        base = wid * _CHUNK
        sems = (sem0, sem1)
        hnd = []
        for h in range(2):
            hnd.append(pltpu.async_copy(
                x_hbm.at[pl.ds(base + h * _H, _H)], xv.at[h], sems[h]))
            hnd.append(pltpu.async_copy(
                bid_hbm.at[pl.ds(base + h * _H, _H)], bv.at[h], sems[h]))

        def zb(t, c):
            for q in range(8):
                histv[pl.ds(t * 128 + q * 16, 16)] = zeros16
            return c
        lax.fori_loop(0, _HW // 128, zb, 0)

        for h in range(2):
            hnd[2 * h].wait()
            hnd[2 * h + 1].wait()

            def s1(i, c):
                for q in range(8):
                    o = i * 128 + q * 16
                    xvec = xv[h, pl.ds(o, 16)]
                    bvec = bv[h, pl.ds(o, 16)]
                    m = xvec != 0.0
                    plsc.addupdate_scatter(
                        histv, [bvec * 16 + lanes], ones16, mask=m)
                return c
            lax.fori_loop(0, _H // 128, s1, 0)

        for c in range(_NT):
            pltpu.sync_copy(histv.at[pl.ds(c * 128, 128)], shv3.at[c, wid])
        plsc.subcore_barrier()

        pltpu.sync_copy(shv3.at[wid], colv)
        for j in range(8):
            acc = colv[0, pl.ds(j * 16, 16)]
            for t in range(1, _NT):
                acc = acc + colv[t, pl.ds(j * 16, 16)]
            sumv[pl.ds(j * 16, 16)] = acc
        pltpu.sync_copy(sumv, shv2.at[pl.ds(wid * 128, 128)])
        plsc.subcore_barrier()

        @pl.when(wid == 0)
        def _():
            pltpu.sync_copy(shv2, hv2)
            running = jnp.int32(0)
            found = jnp.int32(0)
            bstar = jnp.int32(0)
            cumbefore = jnp.int32(0)
            for c in range(_NB // 16):
                acc = zeros16
                for j in range(16):
                    acc = acc + plsc.load_gather(
                        hv2, [c * 256 + lanes * 16 + j])
                s = plsc.cumsum(acc)
                cum = s + running
                mask = cum >= _GOAL
                hasn = jnp.max(plsc.all_reduce_population_count(mask))
                ffs = jnp.max(plsc.all_reduce_ffs(mask))
                hit = (found == 0) & (hasn > 0)
                prev = cum - acc
                prevsel = jnp.max(jnp.where(lanes == ffs, prev, jnp.int32(-1)))
                bstar = jnp.where(hit, c * 16 + ffs, bstar)
                cumbefore = jnp.where(hit, prevsel, cumbefore)
                found = jnp.where(hit, jnp.int32(1), found)
                running = jnp.max(cum)
            count_nz = running
            kprime = _GOAL - cumbefore
            prune = count_nz > _GOAL

            @pl.when(prune)
            def _():
                pltpu.sync_copy(perm_hbm.at[bstar], pv)
                pltpu.sync_copy(su_hbm.at[bstar], suv)
                for g in range(_LPB // 8):
                    hs = [pltpu.async_copy(x_hbm.at[pv.at[8 * g + j]],
                                           gv.at[8 * g + j], sem0)
                          for j in range(8)]
                    for h in hs:
                        h.wait()

                kprime_v = zeros16 + kprime

                def s3(i, carry):
                    cum_v, vstar_v, cumat_v = carry
                    xg = gv[i // 8, pl.ds((i % 8) * 16, 16)]
                    m = xg != 0.0
                    cnt_v = plsc.all_reduce_population_count(m)
                    nxt_v = cum_v + cnt_v
                    crossing = (cum_v < kprime_v) & (nxt_v >= kprime_v)
                    vstar_v = jnp.where(crossing, zeros16 + i, vstar_v)
                    cumat_v = jnp.where(crossing, cum_v, cumat_v)
                    return (nxt_v, vstar_v, cumat_v)
                _, vstar_v, cumat_v = lax.fori_loop(
                    0, _BLK // 16, s3, (zeros16, zeros16, zeros16))
                vstar = jnp.max(vstar_v)
                cumat = jnp.max(cumat_v)
                xg = gv[vstar // 8, pl.ds((vstar % 8) * 16, 16)]
                sv = suv[pl.ds(vstar * 16, 16)]
                m = xg != 0.0
                s = plsc.cumsum(jnp.where(m, 1, 0).astype(jnp.int32))
                sel = m & ((s + cumat) == kprime_v)
                kth = jnp.max(jnp.where(sel, sv, jnp.float32(-1.0)))
                kv[...] = jnp.zeros((16,), jnp.float32) + kth
                pltpu.sync_copy(kv, out_hbm)

            @pl.when(jnp.logical_not(prune))
            def _():
                kv[...] = jnp.full((16,), -1.0, jnp.float32)
                pltpu.sync_copy(kv, out_hbm)

    return k(xf, bid, perm3, su2)


def _tc_mask(kth11, x, u2):
    def body(kth_ref, x_ref, u_ref, o_ref):
        kth = kth_ref[0, 0]
        xb = x_ref[...]
        o_ref[...] = jnp.where((xb != 0.0) & (u_ref[...] >= kth), xb, 0.0)

    return pl.pallas_call(
        body,
        out_shape=jax.ShapeDtypeStruct((_ROWS, _COLS), jnp.float32),
        in_specs=[
            pl.BlockSpec(memory_space=pltpu.SMEM),
            pl.BlockSpec(memory_space=pltpu.VMEM),
            pl.BlockSpec(memory_space=pltpu.VMEM),
        ],
        out_specs=pl.BlockSpec(memory_space=pltpu.VMEM),
    )(kth11, x, u2)


def kernel(input):
    x = input
    xf = x.reshape(-1)
    kth16 = _sc_select(xf, jnp.asarray(_BID_NP), jnp.asarray(_PERM3_NP),
                       jnp.asarray(_SU2_NP))
    kth11 = kth16[:1].reshape(1, 1)
    return _tc_mask(kth11, x, jnp.asarray(_U2_NP))


# speedup vs baseline: 9.2447x; 1.1083x over previous
import functools
import math

import numpy as np
import jax
import jax.numpy as jnp
from jax import lax
from jax.experimental import pallas as pl
from jax.experimental.pallas import tpu as pltpu
from jax.experimental.pallas import tpu_sc as plsc

_ROWS, _COLS = 64, 8192
_N = _ROWS * _COLS
_GOAL = math.floor(0.05 * _N)
_NT = 16
_CHUNK = _N // _NT
_NB = 128
_BLK = _N // _NB
_LPB = _BLK // 128


def _np_threefry2x32(k1, k2, x0, x1):
    x0 = x0.astype(np.uint32).copy()
    x1 = x1.astype(np.uint32).copy()
    ks = [np.uint32(k1), np.uint32(k2),
          np.uint32(k1) ^ np.uint32(k2) ^ np.uint32(0x1BD11BDA)]
    rot_a = (13, 15, 26, 6)
    rot_b = (17, 29, 16, 24)

    def rounds(x0, x1, rots):
        for r in rots:
            x0 = x0 + x1
            x1 = (x1 << np.uint32(r)) | (x1 >> np.uint32(32 - r))
            x1 = x0 ^ x1
        return x0, x1

    x0 += ks[0]
    x1 += ks[1]
    sched = ((rot_a, 1, 2), (rot_b, 2, 0), (rot_a, 0, 1),
             (rot_b, 1, 2), (rot_a, 2, 0))
    for i, (rots, a, b) in enumerate(sched):
        x0, x1 = rounds(x0, x1, rots)
        x0 += ks[a]
        x1 += ks[b] + np.uint32(i + 1)
    return x0, x1


def _np_uniform_key42(n):
    counts1 = np.zeros(n, np.uint32)
    counts2 = np.arange(n, dtype=np.uint32)
    bits1, bits2 = _np_threefry2x32(0, 42, counts1, counts2)
    bits = bits1 ^ bits2
    float_bits = (bits >> np.uint32(9)) | np.uint32(0x3F800000)
    floats = float_bits.view(np.float32) - np.float32(1.0)
    return np.maximum(np.float32(0.0), floats)


def _build_consts():
    u = _np_uniform_key42(_N)
    order = np.argsort(-u).astype(np.int32)
    su = u[order]
    rank = np.empty(_N, dtype=np.int64)
    rank[order] = np.arange(_N)
    bid = ((rank // _BLK) * 16 + (np.arange(_N) % 16)).astype(np.int32)
    return u, order, su, bid


_U_NP, _PERM_NP, _SU_NP, _BID_NP = _build_consts()
_PERM3_NP = np.ascontiguousarray(_PERM_NP.reshape(_NB, _LPB, 128))
_SU2_NP = np.ascontiguousarray(_SU_NP.reshape(_NB, _BLK))


def _sc_randmax(xf, bid, uf, perm3, su2):
    mesh = plsc.VectorSubcoreMesh(core_axis_name="c", subcore_axis_name="s",
                                  num_cores=1)
    _HW = _NB * 16
    _H = _CHUNK // 2

    @functools.partial(
        pl.kernel,
        out_type=jax.ShapeDtypeStruct((_N,), jnp.float32),
        mesh=mesh,
        compiler_params=pltpu.CompilerParams(needs_layout_passes=False),
        scratch_types=[
            pltpu.VMEM((2, _H), jnp.float32),
            pltpu.VMEM((2, _H), jnp.int32),
            pltpu.VMEM((2, _H), jnp.float32),
            pltpu.VMEM((_HW,), jnp.int32),
            pltpu.VMEM((_NT, 128), jnp.int32),
            pltpu.VMEM((128,), jnp.int32),
            pltpu.VMEM((_HW,), jnp.int32),
            pltpu.VMEM_SHARED((_NT, _NT, 128), jnp.int32),
            pltpu.VMEM_SHARED((_HW,), jnp.int32),
            pltpu.VMEM_SHARED((16,), jnp.float32),
            pltpu.VMEM((_LPB, 128), jnp.int32),
            pltpu.VMEM((_BLK,), jnp.float32),
            pltpu.VMEM((_LPB, 128), jnp.float32),
            pltpu.VMEM((16,), jnp.float32),
            pltpu.SemaphoreType.DMA,
            pltpu.SemaphoreType.DMA,
            pltpu.SemaphoreType.DMA,
        ],
    )
    def k(x_hbm, bid_hbm, u_hbm, perm_hbm, su_hbm, out_hbm,
          xv, bv, uv, histv, colv, sumv, hv2, shv3, shv2, shk, pv, suv, gv, kv,
          sem0, sem1, sem2):
        wid = lax.axis_index("s")
        lanes = lax.iota(jnp.int32, 16)
        zeros16 = jnp.zeros((16,), jnp.int32)
        ones16 = jnp.ones((16,), jnp.int32)

        base = wid * _CHUNK
        sems = (sem0, sem1)
        hnd = []
        for h in range(2):
            hnd.append(pltpu.async_copy(
                x_hbm.at[pl.ds(base + h * _H, _H)], xv.at[h], sems[h]))
            hnd.append(pltpu.async_copy(
                bid_hbm.at[pl.ds(base + h * _H, _H)], bv.at[h], sems[h]))
        uh = [pltpu.async_copy(u_hbm.at[pl.ds(base + h * _H, _H)],
                               uv.at[h], sem2) for h in range(2)]

        def zb(t, c):
            for q in range(8):
                histv[pl.ds(t * 128 + q * 16, 16)] = zeros16
            return c
        lax.fori_loop(0, _HW // 128, zb, 0)

        for h in range(2):
            hnd[2 * h].wait()
            hnd[2 * h + 1].wait()

            def s1(i, c):
                xs = [xv[h, pl.ds(i * 128 + q * 16, 16)] for q in range(8)]
                bs = [bv[h, pl.ds(i * 128 + q * 16, 16)] for q in range(8)]
                ms = [xq != 0.0 for xq in xs]
                for q in range(8):
                    plsc.addupdate_scatter(histv, [bs[q]], ones16, mask=ms[q])
                return c
            lax.fori_loop(0, _H // 128, s1, 0)

        for c in range(_NT):
            pltpu.sync_copy(histv.at[pl.ds(c * 128, 128)], shv3.at[c, wid])
        plsc.subcore_barrier()

        pltpu.sync_copy(shv3.at[wid], colv)
        for j in range(8):
            acc = colv[0, pl.ds(j * 16, 16)]
            for t in range(1, _NT):
                acc = acc + colv[t, pl.ds(j * 16, 16)]
            sumv[pl.ds(j * 16, 16)] = acc
        pltpu.sync_copy(sumv, shv2.at[pl.ds(wid * 128, 128)])
        plsc.subcore_barrier()

        @pl.when(wid == 0)
        def _():
            pltpu.sync_copy(shv2, hv2)
            running = jnp.int32(0)
            found = jnp.int32(0)
            bstar = jnp.int32(0)
            cumbefore = jnp.int32(0)
            for c in range(_NB // 16):
                acc = zeros16
                for j in range(16):
                    acc = acc + plsc.load_gather(
                        hv2, [c * 256 + lanes * 16 + j])
                s = plsc.cumsum(acc)
                cum = s + running
                mask = cum >= _GOAL
                hasn = jnp.max(plsc.all_reduce_population_count(mask))
                ffs = jnp.max(plsc.all_reduce_ffs(mask))
                hit = (found == 0) & (hasn > 0)
                prev = cum - acc
                prevsel = jnp.max(jnp.where(lanes == ffs, prev, jnp.int32(-1)))
                bstar = jnp.where(hit, c * 16 + ffs, bstar)
                cumbefore = jnp.where(hit, prevsel, cumbefore)
                found = jnp.where(hit, jnp.int32(1), found)
                running = jnp.max(cum)
            count_nz = running
            kprime = _GOAL - cumbefore
            prune = count_nz > _GOAL

            @pl.when(prune)
            def _():
                pltpu.sync_copy(perm_hbm.at[bstar], pv)
                pltpu.sync_copy(su_hbm.at[bstar], suv)
                for g in range(_LPB // 8):
                    hs = [pltpu.async_copy(x_hbm.at[pv.at[8 * g + j]],
                                           gv.at[8 * g + j], sem0)
                          for j in range(8)]
                    for hh in hs:
                        hh.wait()

                kprime_v = zeros16 + kprime

                def s3(i, carry):
                    cum_v, vstar_v, cumat_v = carry
                    xg = gv[i // 8, pl.ds((i % 8) * 16, 16)]
                    m = xg != 0.0
                    cnt_v = plsc.all_reduce_population_count(m)
                    nxt_v = cum_v + cnt_v
                    crossing = (cum_v < kprime_v) & (nxt_v >= kprime_v)
                    vstar_v = jnp.where(crossing, zeros16 + i, vstar_v)
                    cumat_v = jnp.where(crossing, cum_v, cumat_v)
                    return (nxt_v, vstar_v, cumat_v)
                _, vstar_v, cumat_v = lax.fori_loop(
                    0, _BLK // 16, s3, (zeros16, zeros16, zeros16))
                vstar = jnp.max(vstar_v)
                cumat = jnp.max(cumat_v)
                xg = gv[vstar // 8, pl.ds((vstar % 8) * 16, 16)]
                sv = suv[pl.ds(vstar * 16, 16)]
                m = xg != 0.0
                s = plsc.cumsum(jnp.where(m, 1, 0).astype(jnp.int32))
                sel = m & ((s + cumat) == kprime_v)
                kth = jnp.max(jnp.where(sel, sv, jnp.float32(-1.0)))
                kv[...] = jnp.zeros((16,), jnp.float32) + kth
                pltpu.sync_copy(kv, shk)

            @pl.when(jnp.logical_not(prune))
            def _():
                kv[...] = jnp.full((16,), -1.0, jnp.float32)
                pltpu.sync_copy(kv, shk)

        plsc.subcore_barrier()

        pltpu.sync_copy(shk, kv)
        kth_v = kv[...]
        zf16 = jnp.zeros((16,), jnp.float32)
        for h in range(2):
            uh[h].wait()

            def s4(i, c):
                xs = [xv[h, pl.ds(i * 128 + q * 16, 16)] for q in range(8)]
                us = [uv[h, pl.ds(i * 128 + q * 16, 16)] for q in range(8)]
                rs = [jnp.where((xs[q] != 0.0) & (us[q] >= kth_v), xs[q], zf16)
                      for q in range(8)]
                for q in range(8):
                    xv[h, pl.ds(i * 128 + q * 16, 16)] = rs[q]
                return c
            lax.fori_loop(0, _H // 128, s4, 0)
            pltpu.sync_copy(xv.at[h], out_hbm.at[pl.ds(base + h * _H, _H)])

    return k(xf, bid, uf, perm3, su2)


def kernel(input):
    x = input
    xf = x.reshape(-1)
    out = _sc_randmax(xf, jnp.asarray(_BID_NP), jnp.asarray(_U_NP),
                      jnp.asarray(_PERM3_NP), jnp.asarray(_SU2_NP))
    return out.reshape(_ROWS, _COLS)


# speedup vs baseline: 11.0445x; 1.1947x over previous
import functools
import math

import numpy as np
import jax
import jax.numpy as jnp
from jax import lax
from jax.experimental import pallas as pl
from jax.experimental.pallas import tpu as pltpu
from jax.experimental.pallas import tpu_sc as plsc

_ROWS, _COLS = 64, 8192
_N = _ROWS * _COLS
_GOAL = math.floor(0.05 * _N)
_NT = 16
_CHUNK = _N // _NT
_NB = 128
_BLK = _N // _NB
_LPB = _BLK // 128


def _np_threefry2x32(k1, k2, x0, x1):
    x0 = x0.astype(np.uint32).copy()
    x1 = x1.astype(np.uint32).copy()
    ks = [np.uint32(k1), np.uint32(k2),
          np.uint32(k1) ^ np.uint32(k2) ^ np.uint32(0x1BD11BDA)]
    rot_a = (13, 15, 26, 6)
    rot_b = (17, 29, 16, 24)

    def rounds(x0, x1, rots):
        for r in rots:
            x0 = x0 + x1
            x1 = (x1 << np.uint32(r)) | (x1 >> np.uint32(32 - r))
            x1 = x0 ^ x1
        return x0, x1

    x0 += ks[0]
    x1 += ks[1]
    sched = ((rot_a, 1, 2), (rot_b, 2, 0), (rot_a, 0, 1),
             (rot_b, 1, 2), (rot_a, 2, 0))
    for i, (rots, a, b) in enumerate(sched):
        x0, x1 = rounds(x0, x1, rots)
        x0 += ks[a]
        x1 += ks[b] + np.uint32(i + 1)
    return x0, x1


def _np_uniform_key42(n):
    counts1 = np.zeros(n, np.uint32)
    counts2 = np.arange(n, dtype=np.uint32)
    bits1, bits2 = _np_threefry2x32(0, 42, counts1, counts2)
    bits = bits1 ^ bits2
    float_bits = (bits >> np.uint32(9)) | np.uint32(0x3F800000)
    floats = float_bits.view(np.float32) - np.float32(1.0)
    return np.maximum(np.float32(0.0), floats)


def _build_consts():
    u = _np_uniform_key42(_N)
    order = np.argsort(-u).astype(np.int32)
    su = u[order]
    rank = np.empty(_N, dtype=np.int64)
    rank[order] = np.arange(_N)
    bid = ((rank // _BLK) * 16 + (np.arange(_N) % 16)).astype(np.int32)
    return u, order, su, bid


_U_NP, _PERM_NP, _SU_NP, _BID_NP = _build_consts()
_U2_NP = np.ascontiguousarray(_U_NP.reshape(_ROWS, _COLS))
_PERM3_NP = np.ascontiguousarray(_PERM_NP.reshape(_NB, _LPB, 128))
_SU2_NP = np.ascontiguousarray(_SU_NP.reshape(_NB, _BLK))


def _sc_select(xf, bid, perm3, su2):
    mesh = plsc.VectorSubcoreMesh(core_axis_name="c", subcore_axis_name="s",
                                  num_cores=1)

    _HW = _NB * 16
    _H = _CHUNK // 2

    @functools.partial(
        pl.kernel,
        out_type=jax.ShapeDtypeStruct((16,), jnp.float32),
        mesh=mesh,
        compiler_params=pltpu.CompilerParams(needs_layout_passes=False),
        scratch_types=[
            pltpu.VMEM((2, _H), jnp.float32),
            pltpu.VMEM((2, _H), jnp.int32),
            pltpu.VMEM((_HW,), jnp.int32),
            pltpu.VMEM((_NT, 128), jnp.int32),
            pltpu.VMEM((128,), jnp.int32),
            pltpu.VMEM((_HW,), jnp.int32),
            pltpu.VMEM_SHARED((_NT, _HW), jnp.int32),
            pltpu.VMEM_SHARED((_HW,), jnp.int32),
            pltpu.VMEM((_LPB, 128), jnp.int32),
            pltpu.VMEM((_BLK,), jnp.float32),
            pltpu.VMEM((_LPB, 128), jnp.float32),
            pltpu.VMEM((16,), jnp.float32),
            pltpu.SemaphoreType.DMA,
            pltpu.SemaphoreType.DMA,
        ],
    )
    def k(x_hbm, bid_hbm, perm_hbm, su_hbm, out_hbm,
          xv, bv, histv, colv, sumv, hv2, shv, shv2, pv, suv, gv, kv,
          sem0, sem1):
        wid = lax.axis_index("s")
        lanes = lax.iota(jnp.int32, 16)
        zeros16 = jnp.zeros((16,), jnp.int32)
        ones16 = jnp.ones((16,), jnp.int32)

        base = wid * _CHUNK
        sems = (sem0, sem1)
        hnd = []
        for h in range(2):
            hnd.append(pltpu.async_copy(
                x_hbm.at[pl.ds(base + h * _H, _H)], xv.at[h], sems[h]))
            hnd.append(pltpu.async_copy(
                bid_hbm.at[pl.ds(base + h * _H, _H)], bv.at[h], sems[h]))

        def zb(t, c):
            for q in range(8):
                histv[pl.ds(t * 128 + q * 16, 16)] = zeros16
            return c
        lax.fori_loop(0, _HW // 128, zb, 0)

        for h in range(2):
            hnd[2 * h].wait()
            hnd[2 * h + 1].wait()

            def s1(i, c):
                xs = [xv[h, pl.ds(i * 128 + q * 16, 16)] for q in range(8)]
                bs = [bv[h, pl.ds(i * 128 + q * 16, 16)] for q in range(8)]
                ms = [xq != 0.0 for xq in xs]
                for q in range(8):
                    plsc.addupdate_scatter(histv, [bs[q]], ones16, mask=ms[q])
                return c
            lax.fori_loop(0, _H // 128, s1, 0)

        pltpu.sync_copy(histv, shv.at[wid])
        plsc.subcore_barrier()

        pltpu.sync_copy(shv.at[:, pl.ds(wid * 128, 128)], colv)
        for j in range(8):
            acc = colv[0, pl.ds(j * 16, 16)]
            for t in range(1, _NT):
                acc = acc + colv[t, pl.ds(j * 16, 16)]
            sumv[pl.ds(j * 16, 16)] = acc
        pltpu.sync_copy(sumv, shv2.at[pl.ds(wid * 128, 128)])
        plsc.subcore_barrier()

        @pl.when(wid == 0)
        def _():
            pltpu.sync_copy(shv2, hv2)
            running = jnp.int32(0)
            found = jnp.int32(0)
            bstar = jnp.int32(0)
            cumbefore = jnp.int32(0)
            for c in range(_NB // 16):
                acc = zeros16
                for j in range(16):
                    acc = acc + plsc.load_gather(
                        hv2, [c * 256 + lanes * 16 + j])
                s = plsc.cumsum(acc)
                cum = s + running
                mask = cum >= _GOAL
                hasn = jnp.max(plsc.all_reduce_population_count(mask))
                ffs = jnp.max(plsc.all_reduce_ffs(mask))
                hit = (found == 0) & (hasn > 0)
                prev = cum - acc
                prevsel = jnp.max(jnp.where(lanes == ffs, prev, jnp.int32(-1)))
                bstar = jnp.where(hit, c * 16 + ffs, bstar)
                cumbefore = jnp.where(hit, prevsel, cumbefore)
                found = jnp.where(hit, jnp.int32(1), found)
                running = jnp.max(cum)
            count_nz = running
            kprime = _GOAL - cumbefore
            prune = count_nz > _GOAL

            @pl.when(prune)
            def _():
                pltpu.sync_copy(perm_hbm.at[bstar], pv)
                pltpu.sync_copy(su_hbm.at[bstar], suv)
                for g in range(_LPB // 16):
                    hs = [pltpu.async_copy(x_hbm.at[pv.at[16 * g + j]],
                                           gv.at[16 * g + j], sem0)
                          for j in range(16)]
                    for h in hs:
                        h.wait()

                kprime_v = zeros16 + kprime

                def s3(i, carry):
                    cum_v, vstar_v, cumat_v = carry
                    for q in range(4):
                        t = i * 4 + q
                        xg = gv[(i * 4 + q) // 8, pl.ds(((i * 4 + q) % 8) * 16, 16)]
                        m = xg != 0.0
                        cnt_v = plsc.all_reduce_population_count(m)
                        nxt_v = cum_v + cnt_v
                        crossing = (cum_v < kprime_v) & (nxt_v >= kprime_v)
                        vstar_v = jnp.where(crossing, zeros16 + t, vstar_v)
                        cumat_v = jnp.where(crossing, cum_v, cumat_v)
                        cum_v = nxt_v
                    return (cum_v, vstar_v, cumat_v)
                _, vstar_v, cumat_v = lax.fori_loop(
                    0, _BLK // 64, s3, (zeros16, zeros16, zeros16))
                vstar = jnp.max(vstar_v)
                cumat = jnp.max(cumat_v)
                xg = gv[vstar // 8, pl.ds((vstar % 8) * 16, 16)]
                sv = suv[pl.ds(vstar * 16, 16)]
                m = xg != 0.0
                s = plsc.cumsum(jnp.where(m, 1, 0).astype(jnp.int32))
                sel = m & ((s + cumat) == kprime_v)
                kth = jnp.max(jnp.where(sel, sv, jnp.float32(-1.0)))
                kv[...] = jnp.zeros((16,), jnp.float32) + kth
                pltpu.sync_copy(kv, out_hbm)

            @pl.when(jnp.logical_not(prune))
            def _():
                kv[...] = jnp.full((16,), -1.0, jnp.float32)
                pltpu.sync_copy(kv, out_hbm)

    return k(xf, bid, perm3, su2)


def _tc_mask(kth11, x, u2):
    def body(kth_ref, x_ref, u_ref, o_ref):
        kth = kth_ref[0, 0]
        xb = x_ref[...]
        o_ref[...] = jnp.where((xb != 0.0) & (u_ref[...] >= kth), xb, 0.0)

    return pl.pallas_call(
        body,
        out_shape=jax.ShapeDtypeStruct((_ROWS, _COLS), jnp.float32),
        in_specs=[
            pl.BlockSpec(memory_space=pltpu.SMEM),
            pl.BlockSpec(memory_space=pltpu.VMEM),
            pl.BlockSpec(memory_space=pltpu.VMEM),
        ],
        out_specs=pl.BlockSpec(memory_space=pltpu.VMEM),
    )(kth11, x, u2)


def kernel(input):
    x = input
    xf = x.reshape(-1)
    kth16 = _sc_select(xf, jnp.asarray(_BID_NP), jnp.asarray(_PERM3_NP),
                       jnp.asarray(_SU2_NP))
    kth11 = kth16[:1].reshape(1, 1)
    return _tc_mask(kth11, x, jnp.asarray(_U2_NP))


# speedup vs baseline: 12.7107x; 1.1509x over previous
import functools
import math

import numpy as np
import jax
import jax.numpy as jnp
from jax import lax
from jax.experimental import pallas as pl
from jax.experimental.pallas import tpu as pltpu
from jax.experimental.pallas import tpu_sc as plsc

_ROWS, _COLS = 64, 8192
_N = _ROWS * _COLS
_GOAL = math.floor(0.05 * _N)
_NT = 16
_CHUNK = _N // _NT
_NB = 128
_BLK = _N // _NB
_LPB = _BLK // 128


def _np_threefry2x32(k1, k2, x0, x1):
    x0 = x0.astype(np.uint32).copy()
    x1 = x1.astype(np.uint32).copy()
    ks = [np.uint32(k1), np.uint32(k2),
          np.uint32(k1) ^ np.uint32(k2) ^ np.uint32(0x1BD11BDA)]
    rot_a = (13, 15, 26, 6)
    rot_b = (17, 29, 16, 24)

    def rounds(x0, x1, rots):
        for r in rots:
            x0 = x0 + x1
            x1 = (x1 << np.uint32(r)) | (x1 >> np.uint32(32 - r))
            x1 = x0 ^ x1
        return x0, x1

    x0 += ks[0]
    x1 += ks[1]
    sched = ((rot_a, 1, 2), (rot_b, 2, 0), (rot_a, 0, 1),
             (rot_b, 1, 2), (rot_a, 2, 0))
    for i, (rots, a, b) in enumerate(sched):
        x0, x1 = rounds(x0, x1, rots)
        x0 += ks[a]
        x1 += ks[b] + np.uint32(i + 1)
    return x0, x1


def _np_uniform_key42(n):
    counts1 = np.zeros(n, np.uint32)
    counts2 = np.arange(n, dtype=np.uint32)
    bits1, bits2 = _np_threefry2x32(0, 42, counts1, counts2)
    bits = bits1 ^ bits2
    float_bits = (bits >> np.uint32(9)) | np.uint32(0x3F800000)
    floats = float_bits.view(np.float32) - np.float32(1.0)
    return np.maximum(np.float32(0.0), floats)


def _build_consts():
    u = _np_uniform_key42(_N)
    order = np.argsort(-u).astype(np.int32)
    su = u[order]
    rank = np.empty(_N, dtype=np.int64)
    rank[order] = np.arange(_N)
    bid = ((rank // _BLK) * 16 + (np.arange(_N) % 16)).astype(np.int32)
    return u, order, su, bid


_U_NP, _PERM_NP, _SU_NP, _BID_NP = _build_consts()
_U2_NP = np.ascontiguousarray(_U_NP.reshape(_ROWS, _COLS))
_PERM3_NP = np.ascontiguousarray(_PERM_NP.reshape(_NB, _LPB, 128))
_SU2_NP = np.ascontiguousarray(_SU_NP.reshape(_NB, _BLK))


def _sc_select(xf, bid, perm3, su2):
    mesh = plsc.VectorSubcoreMesh(core_axis_name="c", subcore_axis_name="s",
                                  num_cores=1)

    _HW = _NB * 16
    _H = _CHUNK // 2
    _BEXP = (_GOAL - 1) // _BLK

    @functools.partial(
        pl.kernel,
        out_type=jax.ShapeDtypeStruct((16,), jnp.float32),
        mesh=mesh,
        compiler_params=pltpu.CompilerParams(needs_layout_passes=False),
        scratch_types=[
            pltpu.VMEM((2, _H), jnp.float32),
            pltpu.VMEM((2, _H), jnp.int32),
            pltpu.VMEM((_HW,), jnp.int32),
            pltpu.VMEM((_NT, 128), jnp.int32),
            pltpu.VMEM((128,), jnp.int32),
            pltpu.VMEM((_HW,), jnp.int32),
            pltpu.VMEM_SHARED((_NT, _HW), jnp.int32),
            pltpu.VMEM_SHARED((_HW,), jnp.int32),
            pltpu.VMEM((_LPB, 128), jnp.int32),
            pltpu.VMEM((_BLK,), jnp.float32),
            pltpu.VMEM((_LPB, 128), jnp.float32),
            pltpu.VMEM((16,), jnp.float32),
            pltpu.SemaphoreType.DMA,
            pltpu.SemaphoreType.DMA,
            pltpu.SemaphoreType.DMA,
            pltpu.SemaphoreType.DMA,
        ],
    )
    def k(x_hbm, bid_hbm, perm_hbm, su_hbm, out_hbm,
          xv, bv, histv, colv, sumv, hv2, shv, shv2, pv, suv, gv, kv,
          sem0, sem1, sem2, sem3):
        wid = lax.axis_index("s")
        lanes = lax.iota(jnp.int32, 16)
        zeros16 = jnp.zeros((16,), jnp.int32)
        ones16 = jnp.ones((16,), jnp.int32)

        base = wid * _CHUNK
        sems = (sem0, sem1)
        hnd = []
        for h in range(2):
            hnd.append(pltpu.async_copy(
                x_hbm.at[pl.ds(base + h * _H, _H)], xv.at[h], sems[h]))
            hnd.append(pltpu.async_copy(
                bid_hbm.at[pl.ds(base + h * _H, _H)], bv.at[h], sems[h]))

        @pl.when(wid == 0)
        def _():
            pltpu.sync_copy(perm_hbm.at[_BEXP], pv)
            for j in range(_LPB):
                pltpu.async_copy(x_hbm.at[pv.at[j]], gv.at[j], sem2)
            pltpu.async_copy(su_hbm.at[_BEXP], suv, sem3)

        def zb(t, c):
            for q in range(8):
                histv[pl.ds(t * 128 + q * 16, 16)] = zeros16
            return c
        lax.fori_loop(0, _HW // 128, zb, 0)

        for h in range(2):
            hnd[2 * h].wait()
            hnd[2 * h + 1].wait()

            def s1(i, c):
                xs = [xv[h, pl.ds(i * 128 + q * 16, 16)] for q in range(8)]
                bs = [bv[h, pl.ds(i * 128 + q * 16, 16)] for q in range(8)]
                ms = [xq != 0.0 for xq in xs]
                for q in range(8):
                    plsc.addupdate_scatter(histv, [bs[q]], ones16, mask=ms[q])
                return c
            lax.fori_loop(0, _H // 128, s1, 0)

        pltpu.sync_copy(histv, shv.at[wid])
        plsc.subcore_barrier()

        pltpu.sync_copy(shv.at[:, pl.ds(wid * 128, 128)], colv)
        for j in range(8):
            acc = colv[0, pl.ds(j * 16, 16)]
            for t in range(1, _NT):
                acc = acc + colv[t, pl.ds(j * 16, 16)]
            sumv[pl.ds(j * 16, 16)] = acc
        pltpu.sync_copy(sumv, shv2.at[pl.ds(wid * 128, 128)])
        plsc.subcore_barrier()

        @pl.when(wid == 0)
        def _():
            pltpu.sync_copy(shv2, hv2)
            running = jnp.int32(0)
            found = jnp.int32(0)
            bstar = jnp.int32(0)
            cumbefore = jnp.int32(0)
            for c in range(_NB // 16):
                acc = zeros16
                for j in range(16):
                    acc = acc + plsc.load_gather(
                        hv2, [c * 256 + lanes * 16 + j])
                s = plsc.cumsum(acc)
                cum = s + running
                mask = cum >= _GOAL
                hasn = jnp.max(plsc.all_reduce_population_count(mask))
                ffs = jnp.max(plsc.all_reduce_ffs(mask))
                hit = (found == 0) & (hasn > 0)
                prev = cum - acc
                prevsel = jnp.max(jnp.where(lanes == ffs, prev, jnp.int32(-1)))
                bstar = jnp.where(hit, c * 16 + ffs, bstar)
                cumbefore = jnp.where(hit, prevsel, cumbefore)
                found = jnp.where(hit, jnp.int32(1), found)
                running = jnp.max(cum)
            count_nz = running
            kprime = _GOAL - cumbefore
            prune = count_nz > _GOAL

            for j in range(_LPB):
                pltpu.make_async_copy(x_hbm.at[pv.at[j]], gv.at[j],
                                      sem2).wait()
            pltpu.make_async_copy(su_hbm.at[_BEXP], suv, sem3).wait()

            @pl.when(prune & (bstar != _BEXP))
            def _():
                pltpu.sync_copy(perm_hbm.at[bstar], pv)
                pltpu.sync_copy(su_hbm.at[bstar], suv)

                def rg(j, c):
                    pltpu.async_copy(x_hbm.at[pv.at[j]], gv.at[j],
                                     sem2).wait()
                    return c
                lax.fori_loop(0, _LPB, rg, 0)

            @pl.when(prune)
            def _():
                kprime_v = zeros16 + kprime

                def s3(i, carry):
                    cum_v, vstar_v, cumat_v = carry
                    for q in range(4):
                        t = i * 4 + q
                        xg = gv[(i * 4 + q) // 8, pl.ds(((i * 4 + q) % 8) * 16, 16)]
                        m = xg != 0.0
                        cnt_v = plsc.all_reduce_population_count(m)
                        nxt_v = cum_v + cnt_v
                        crossing = (cum_v < kprime_v) & (nxt_v >= kprime_v)
                        vstar_v = jnp.where(crossing, zeros16 + t, vstar_v)
                        cumat_v = jnp.where(crossing, cum_v, cumat_v)
                        cum_v = nxt_v
                    return (cum_v, vstar_v, cumat_v)
                _, vstar_v, cumat_v = lax.fori_loop(
                    0, _BLK // 64, s3, (zeros16, zeros16, zeros16))
                vstar = jnp.max(vstar_v)
                cumat = jnp.max(cumat_v)
                xg = gv[vstar // 8, pl.ds((vstar % 8) * 16, 16)]
                sv = suv[pl.ds(vstar * 16, 16)]
                m = xg != 0.0
                s = plsc.cumsum(jnp.where(m, 1, 0).astype(jnp.int32))
                sel = m & ((s + cumat) == kprime_v)
                kth = jnp.max(jnp.where(sel, sv, jnp.float32(-1.0)))
                kv[...] = jnp.zeros((16,), jnp.float32) + kth
                pltpu.sync_copy(kv, out_hbm)

            @pl.when(jnp.logical_not(prune))
            def _():
                kv[...] = jnp.full((16,), -1.0, jnp.float32)
                pltpu.sync_copy(kv, out_hbm)

    return k(xf, bid, perm3, su2)


def _tc_mask(kth11, x, u2):
    def body(kth_ref, x_ref, u_ref, o_ref):
        kth = kth_ref[0, 0]
        xb = x_ref[...]
        o_ref[...] = jnp.where((xb != 0.0) & (u_ref[...] >= kth), xb, 0.0)

    return pl.pallas_call(
        body,
        out_shape=jax.ShapeDtypeStruct((_ROWS, _COLS), jnp.float32),
        in_specs=[
            pl.BlockSpec(memory_space=pltpu.SMEM),
            pl.BlockSpec(memory_space=pltpu.VMEM),
            pl.BlockSpec(memory_space=pltpu.VMEM),
        ],
        out_specs=pl.BlockSpec(memory_space=pltpu.VMEM),
    )(kth11, x, u2)


def kernel(input):
    x = input
    xf = x.reshape(-1)
    kth16 = _sc_select(xf, jnp.asarray(_BID_NP), jnp.asarray(_PERM3_NP),
                       jnp.asarray(_SU2_NP))
    kth11 = kth16[:1].reshape(1, 1)
    return _tc_mask(kth11, x, jnp.asarray(_U2_NP))
